# baseline (speedup 1.0000x reference)
import jax
import jax.numpy as jnp
from jax.experimental import pallas as pl


def _masked_matvec_kernel(x_ref, m_ref, w_ref, b_ref, o_ref):
    x = x_ref[...]
    w = w_ref[...]
    y = jax.lax.dot_general(
        x, w, (((1,), (1,)), ((), ())), preferred_element_type=jnp.float32
    )
    y = y[:, 0] + b_ref[0, 0]
    o_ref[0, 0, :] = jnp.where(m_ref[0, 0, :] > 0, y, 0.0)


def kernel(embs, masks, W, b):
    B, N, D = embs.shape
    R = B * N
    ROWS = 2048
    G = R // ROWS

    x = embs.reshape(R, D)
    m = masks.reshape(R).astype(jnp.float32).reshape(G, 1, ROWS)
    b_arr = b.reshape(1, 1).astype(jnp.float32)

    out = pl.pallas_call(
        _masked_matvec_kernel,
        grid=(G,),
        in_specs=[
            pl.BlockSpec((ROWS, D), lambda i: (i, 0)),
            pl.BlockSpec((1, 1, ROWS), lambda i: (i, 0, 0)),
            pl.BlockSpec((1, D), lambda i: (0, 0)),
            pl.BlockSpec((1, 1), lambda i: (0, 0)),
        ],
        out_specs=pl.BlockSpec((1, 1, ROWS), lambda i: (i, 0, 0)),
        out_shape=jax.ShapeDtypeStruct((G, 1, ROWS), jnp.float32),
    )(x, m, W.astype(jnp.float32), b_arr)

    return out.reshape(B, N)


# speedup vs baseline: 1.4165x; 1.4165x over previous
import jax
import jax.numpy as jnp
from jax.experimental import pallas as pl


def _masked_matvec_kernel(x_ref, m_ref, w_ref, b_ref, o_ref):
    x = x_ref[...]
    w = w_ref[...]
    y = jax.lax.dot_general(
        w, x, (((1,), (1,)), ((), ())), preferred_element_type=jnp.float32
    )
    y = y[0] + b_ref[0, 0]
    o_ref[0, 0, :] = jnp.where(m_ref[0, 0, :] > 0, y, 0.0)


def kernel(embs, masks, W, b):
    B, N, D = embs.shape
    R = B * N
    ROWS = 2048
    G = R // ROWS

    x = embs.reshape(R, D)
    m = masks.reshape(R).astype(jnp.float32).reshape(G, 1, ROWS)
    b_arr = b.reshape(1, 1).astype(jnp.float32)

    out = pl.pallas_call(
        _masked_matvec_kernel,
        grid=(G,),
        in_specs=[
            pl.BlockSpec((ROWS, D), lambda i: (i, 0)),
            pl.BlockSpec((1, 1, ROWS), lambda i: (i, 0, 0)),
            pl.BlockSpec((1, D), lambda i: (0, 0)),
            pl.BlockSpec((1, 1), lambda i: (0, 0)),
        ],
        out_specs=pl.BlockSpec((1, 1, ROWS), lambda i: (i, 0, 0)),
        out_shape=jax.ShapeDtypeStruct((G, 1, ROWS), jnp.float32),
    )(x, m, W.astype(jnp.float32), b_arr)

    return out.reshape(B, N)


# speedup vs baseline: 1.9839x; 1.4006x over previous
import jax
import jax.numpy as jnp
from jax.experimental import pallas as pl


def _masked_matvec_kernel(x_ref, m_ref, w_ref, b_ref, o_ref):
    x = x_ref[...]
    w = w_ref[...]
    y = jax.lax.dot_general(
        w, x, (((1,), (1,)), ((), ())), preferred_element_type=jnp.float32
    )
    y = y[0] + b_ref[0, 0]
    o_ref[0, 0, :] = jnp.where(m_ref[0, 0, :] > 0, y, 0.0)


def kernel(embs, masks, W, b):
    B, N, D = embs.shape
    R = B * N
    ROWS = 4096
    G = R // ROWS

    x = embs.reshape(R, D)
    m = masks.reshape(R).astype(jnp.float32).reshape(G, 1, ROWS)
    b_arr = b.reshape(1, 1).astype(jnp.float32)

    out = pl.pallas_call(
        _masked_matvec_kernel,
        grid=(G,),
        in_specs=[
            pl.BlockSpec((ROWS, D), lambda i: (i, 0)),
            pl.BlockSpec((1, 1, ROWS), lambda i: (i, 0, 0)),
            pl.BlockSpec((1, D), lambda i: (0, 0)),
            pl.BlockSpec((1, 1), lambda i: (0, 0)),
        ],
        out_specs=pl.BlockSpec((1, 1, ROWS), lambda i: (i, 0, 0)),
        out_shape=jax.ShapeDtypeStruct((G, 1, ROWS), jnp.float32),
    )(x, m, W.astype(jnp.float32), b_arr)

    return out.reshape(B, N)


# speedup vs baseline: 2.6095x; 1.3153x over previous
import jax
import jax.numpy as jnp
from jax.experimental import pallas as pl


def _masked_matvec_kernel(x_ref, m_ref, w_ref, b_ref, o_ref):
    x = x_ref[...]
    w = w_ref[...]
    y = jax.lax.dot_general(
        w, x, (((1,), (1,)), ((), ())), preferred_element_type=jnp.float32
    )
    y = y[0] + b_ref[0, 0]
    o_ref[0, 0, :] = jnp.where(m_ref[0, 0, :] > 0, y, 0.0)


def kernel(embs, masks, W, b):
    B, N, D = embs.shape
    R = B * N
    ROWS = 8192
    G = R // ROWS

    x = embs.reshape(R, D)
    m = masks.reshape(R).astype(jnp.float32).reshape(G, 1, ROWS)
    b_arr = b.reshape(1, 1).astype(jnp.float32)

    out = pl.pallas_call(
        _masked_matvec_kernel,
        grid=(G,),
        in_specs=[
            pl.BlockSpec((ROWS, D), lambda i: (i, 0)),
            pl.BlockSpec((1, 1, ROWS), lambda i: (i, 0, 0)),
            pl.BlockSpec((1, D), lambda i: (0, 0)),
            pl.BlockSpec((1, 1), lambda i: (0, 0)),
        ],
        out_specs=pl.BlockSpec((1, 1, ROWS), lambda i: (i, 0, 0)),
        out_shape=jax.ShapeDtypeStruct((G, 1, ROWS), jnp.float32),
    )(x, m, W.astype(jnp.float32), b_arr)

    return out.reshape(B, N)


# speedup vs baseline: 2.8316x; 1.0851x over previous
import jax
import jax.numpy as jnp
from jax.experimental import pallas as pl


def _masked_matvec_kernel(x_ref, m_ref, w_ref, b_ref, o_ref):
    x = x_ref[...]
    w = w_ref[...]
    y = jax.lax.dot_general(
        w, x, (((1,), (1,)), ((), ())), preferred_element_type=jnp.float32
    )
    y = y[0] + b_ref[0, 0]
    o_ref[0, 0, :] = jnp.where(m_ref[0, 0, :] > 0, y, 0.0)


def kernel(embs, masks, W, b):
    B, N, D = embs.shape
    R = B * N
    ROWS = 16384
    G = R // ROWS

    x = embs.reshape(R, D)
    m = masks.reshape(R).astype(jnp.float32).reshape(G, 1, ROWS)
    b_arr = b.reshape(1, 1).astype(jnp.float32)

    out = pl.pallas_call(
        _masked_matvec_kernel,
        grid=(G,),
        in_specs=[
            pl.BlockSpec((ROWS, D), lambda i: (i, 0)),
            pl.BlockSpec((1, 1, ROWS), lambda i: (i, 0, 0)),
            pl.BlockSpec((1, D), lambda i: (0, 0)),
            pl.BlockSpec((1, 1), lambda i: (0, 0)),
        ],
        out_specs=pl.BlockSpec((1, 1, ROWS), lambda i: (i, 0, 0)),
        out_shape=jax.ShapeDtypeStruct((G, 1, ROWS), jnp.float32),
    )(x, m, W.astype(jnp.float32), b_arr)

    return out.reshape(B, N)
